# baseline (speedup 1.0000x reference)
import functools
import math

import jax
import jax.numpy as jnp
import numpy as np
from jax import lax
from jax.experimental import pallas as pl
from jax.experimental.pallas import tpu as pltpu
from jax.experimental.pallas import tpu_sc as plsc

D = 256
L = 2
H = 4
DH = D // H
DFF = 1024
N = 8
TS = 512
K = 16
TK = 128
NEG = -1e9


def _sinusoid_np(T, d):
    pos = np.arange(T)[:, None].astype(np.float32)
    i = np.arange(d)[None, :].astype(np.float32)
    angle = pos / np.power(10000.0, (2.0 * np.floor(i / 2.0)) / d)
    pe = np.zeros((T, d), dtype=np.float32)
    pe[:, 0::2] = np.sin(angle[:, 0::2])
    pe[:, 1::2] = np.cos(angle[:, 1::2])
    return pe


_PE_CTX = _sinusoid_np(TS, D)
_PE_KN8 = np.tile(_sinusoid_np(TK, D), (8, 1))


def _sc_embed_gather(table, idx):
    info = plsc.get_sparse_core_info()
    nw = info.num_cores * info.num_subcores
    b = idx.shape[0]
    b_per_w = b // nw
    ch = max(c for c in range(1, min(128, b_per_w) + 1)
             if b_per_w % c == 0)
    n_ch = b_per_w // ch
    nc = info.num_cores
    NB = min(3, n_ch)
    idx3d = idx.reshape(nw, n_ch, ch)
    mesh = plsc.VectorSubcoreMesh(core_axis_name="c", subcore_axis_name="s")

    @functools.partial(
        pl.kernel,
        mesh=mesh,
        out_type=jax.ShapeDtypeStruct((b, D), jnp.float32),
        scratch_types=[
            pltpu.VMEM((n_ch, ch), jnp.int32),
            pltpu.VMEM((NB, ch, D), jnp.float32),
            pltpu.SemaphoreType.DMA,
            pltpu.SemaphoreType.DMA,
        ],
    )
    def gather_kernel(table_hbm, idx_hbm, out_hbm, idx_v, rows_v, gsem, wsem):
        wid = lax.axis_index("s") * nc + lax.axis_index("c")
        base = wid * b_per_w
        pltpu.sync_copy(idx_hbm.at[wid], idx_v)
        gathers = [None] * n_ch
        writes = [None] * n_ch
        for i in range(min(NB, n_ch)):
            gathers[i] = pltpu.async_copy(
                table_hbm.at[idx_v.at[i]], rows_v.at[i], gsem)
        for i in range(n_ch):
            gathers[i].wait()
            writes[i] = pltpu.async_copy(
                rows_v.at[i % NB], out_hbm.at[pl.ds(base + i * ch, ch)], wsem)
            if i + NB < n_ch:
                writes[i].wait()
                gathers[i + NB] = pltpu.async_copy(
                    table_hbm.at[idx_v.at[i + NB]], rows_v.at[i % NB], gsem)
        for i in range(max(0, n_ch - NB), n_ch):
            writes[i].wait()

    return gather_kernel(table, idx3d)


def _ln(x, g, b):
    mu = jnp.mean(x, axis=-1, keepdims=True)
    m2 = jnp.mean(x * x, axis=-1, keepdims=True)
    var = m2 - mu * mu
    return (x - mu) * lax.rsqrt(var + 1e-5) * g + b


def _encode_body(emb_ref, mrow_ref, mcol_ref, pe_ref,
                 wq_ref, wk_ref, wv_ref, wo_ref,
                 w1_ref, b1_ref, w2_ref, b2_ref,
                 g1_ref, bn1_ref, g2_ref, bn2_ref,
                 enc_ref, pool_ref, *, T, G):
    gt = G * T
    mrow = mrow_ref[0]
    mcol = mcol_ref[0]
    bias = (mrow - 1.0) * 1e9
    x = emb_ref[0] * np.float32(math.sqrt(D)) + pe_ref[...]
    inv_sqrt_dh = np.float32(1.0 / math.sqrt(DH))
    for l in range(L):
        q = jnp.dot(x, wq_ref[l])
        k = jnp.dot(x, wk_ref[l])
        v = jnp.dot(x, wv_ref[l])
        wo = wo_ref[l]
        o_rows = []
        for g in range(G):
            rs = slice(g * T, (g + 1) * T)
            bias_g = bias[g:g + 1, :]
            acc = None
            for h in range(H):
                cs = slice(h * DH, (h + 1) * DH)
                s = lax.dot_general(q[rs, cs], k[rs, cs],
                                    (((1,), (1,)), ((), ()))) * inv_sqrt_dh
                s = s + bias_g
                s = s - jnp.max(s, axis=-1, keepdims=True)
                p = jnp.exp(s)
                denom = jnp.sum(p, axis=-1, keepdims=True)
                oh = jnp.dot(p, v[rs, cs]) / denom
                part = jnp.dot(oh, wo[cs, :])
                acc = part if acc is None else acc + part
            o_rows.append(acc)
        o = jnp.concatenate(o_rows, axis=0) if G > 1 else o_rows[0]
        x = _ln(x + o, g1_ref[l], bn1_ref[l])
        hdn = jnp.maximum(jnp.dot(x, w1_ref[l]) + b1_ref[l], 0.0)
        x = _ln(x + jnp.dot(hdn, w2_ref[l]) + b2_ref[l], g2_ref[l], bn2_ref[l])
    xm = x * mcol
    enc_ref[0] = xm
    for g in range(G):
        seg = xm[g * T:(g + 1) * T, :]
        ssum = jnp.sum(seg, axis=0, keepdims=True)
        ln_g = jnp.maximum(jnp.sum(mrow[g]), 1.0)
        pool_ref[0, g:g + 1, :] = ssum * lax.rsqrt(ln_g * np.float32(D))


def _encode_pallas(emb3d, mrow, mcol, pe_big,
                   Wq, Wk, Wv, Wo, W1, b1, W2, b2, g1, bn1, g2, bn2,
                   *, T, G):
    nblk = emb3d.shape[0]
    gt = G * T
    full = lambda shape: pl.BlockSpec(shape, lambda i: tuple(0 for _ in shape))
    out = pl.pallas_call(
        functools.partial(_encode_body, T=T, G=G),
        grid=(nblk,),
        in_specs=[
            pl.BlockSpec((1, gt, D), lambda i: (i, 0, 0)),
            pl.BlockSpec((1, G, T), lambda i: (i, 0, 0)),
            pl.BlockSpec((1, gt, 1), lambda i: (i, 0, 0)),
            full((gt, D)),
            full((L, D, D)), full((L, D, D)), full((L, D, D)), full((L, D, D)),
            full((L, D, DFF)), full((L, DFF)),
            full((L, DFF, D)), full((L, D)),
            full((L, D)), full((L, D)), full((L, D)), full((L, D)),
        ],
        out_specs=[
            pl.BlockSpec((1, gt, D), lambda i: (i, 0, 0)),
            pl.BlockSpec((1, G, D), lambda i: (i, 0, 0)),
        ],
        out_shape=[
            jax.ShapeDtypeStruct((nblk, gt, D), jnp.float32),
            jax.ShapeDtypeStruct((nblk, G, D), jnp.float32),
        ],
    )(emb3d, mrow, mcol, pe_big, Wq, Wk, Wv, Wo, W1, b1, W2, b2,
      g1, bn1, g2, bn2)
    return out


def _select_body(ids_ref, flag_ref,
                 ku_ref, cu_ref, ckm_ref, ke_ref, kt_ref,
                 attn_ref, cse_ref, stok_ref):
    n = pl.program_id(0)
    ku = ku_ref[0]
    cu = cu_ref[0]
    raw = lax.dot_general(cu, ku, (((1,), (1,)), ((), ())))
    ckm = ckm_ref[0]
    attn_ref[0] = raw * ckm
    masked = jnp.where(ckm > 0.0, raw, NEG)
    mx = jnp.max(masked)
    iota = lax.broadcasted_iota(jnp.int32, (1, K), 1)
    amax = jnp.min(jnp.where(masked >= mx, iota, K))
    sel = jnp.where(flag_ref[0] != 0, ids_ref[n], amax)
    cse_ref[...] = ke_ref[0, pl.ds(sel, 1)]
    stok_ref[0] = kt_ref[0, pl.ds(sel, 1)]


def _select_pallas(cs_ids, use_flag, know_use, ctx_use, ckm, know_enc, know_tok):
    nb = know_use.shape[0]
    grid_spec = pltpu.PrefetchScalarGridSpec(
        num_scalar_prefetch=2,
        grid=(nb,),
        in_specs=[
            pl.BlockSpec((1, K, D), lambda n, i, f: (n, 0, 0)),
            pl.BlockSpec((1, 1, D), lambda n, i, f: (n, 0, 0)),
            pl.BlockSpec((1, 1, K), lambda n, i, f: (n, 0, 0)),
            pl.BlockSpec((1, K, TK, D), lambda n, i, f: (n, 0, 0, 0)),
            pl.BlockSpec((1, K, TK), lambda n, i, f: (n, 0, 0)),
        ],
        out_specs=[
            pl.BlockSpec((1, 1, K), lambda n, i, f: (n, 0, 0)),
            pl.BlockSpec((1, TK, D), lambda n, i, f: (n, 0, 0)),
            pl.BlockSpec((1, 1, TK), lambda n, i, f: (n, 0, 0)),
        ],
    )
    return pl.pallas_call(
        _select_body,
        grid_spec=grid_spec,
        out_shape=[
            jax.ShapeDtypeStruct((nb, 1, K), jnp.float32),
            jax.ShapeDtypeStruct((nb, TK, D), jnp.float32),
            jax.ShapeDtypeStruct((nb, 1, TK), jnp.int32),
        ],
    )(cs_ids, use_flag, know_use, ctx_use, ckm, know_enc, know_tok)


def kernel(src_tokens, know_tokens, ck_mask, cs_ids, use_cs_ids, embed,
           Wq, Wk, Wv, Wo, W1, b1, W2, b2, g1, bn1, g2, bn2):
    src_tokens = src_tokens.astype(jnp.int32)
    know_tokens = know_tokens.astype(jnp.int32)
    kn_splits = [(0, 3), (3, 6), (6, 7), (7, 8)]
    ctx_splits = [(0, 2), (2, 8)]
    ctx_embs = [
        _sc_embed_gather(embed, src_tokens[lo:hi].reshape(-1))
        .reshape(-1, TS, D) for lo, hi in ctx_splits]
    kn_embs = [
        _sc_embed_gather(embed, know_tokens[lo:hi].reshape(-1))
        .reshape(-1, 8 * TK, D) for lo, hi in kn_splits]

    ctx_maskf = (src_tokens != 0).astype(jnp.float32)
    kn_flat = know_tokens.reshape(N * K, TK)
    kn_maskf = (kn_flat != 0).astype(jnp.float32)

    weights = (Wq, Wk, Wv, Wo, W1, b1, W2, b2, g1, bn1, g2, bn2)
    ctx_mrow = ctx_maskf.reshape(N, 1, TS)
    ctx_mcol = ctx_maskf.reshape(N, TS, 1)
    ctx_parts = [
        _encode_pallas(ctx_embs[j], ctx_mrow[lo:hi],
                       ctx_mcol[lo:hi], _PE_CTX,
                       *weights, T=TS, G=1)
        for j, (lo, hi) in enumerate(ctx_splits)]
    kn_mrow = kn_maskf.reshape(16, 8, TK)
    kn_mcol = kn_maskf.reshape(16, 8 * TK, 1)
    kn_parts = [
        _encode_pallas(kn_embs[j], kn_mrow[2 * lo:2 * hi],
                       kn_mcol[2 * lo:2 * hi], _PE_KN8,
                       *weights, T=TK, G=8)
        for j, (lo, hi) in enumerate(kn_splits)]
    ctx_pool = jnp.concatenate([p[1] for p in ctx_parts], axis=0)

    ctx_use = ctx_pool.reshape(N, 1, D)
    ckm = ck_mask.astype(jnp.float32).reshape(N, 1, K)
    flag = jnp.asarray(use_cs_ids, jnp.int32).reshape(1)
    ids = cs_ids.astype(jnp.int32)
    know_tok4 = know_tokens.reshape(N, K, TK)

    sel_parts = [
        _select_pallas(
            ids[lo:hi], flag,
            kn_parts[j][1].reshape(-1, K, D),
            ctx_use[lo:hi],
            ckm[lo:hi],
            kn_parts[j][0].reshape(-1, K, TK, D),
            know_tok4[lo:hi])
        for j, (lo, hi) in enumerate(kn_splits)]
    ck_attn3 = jnp.concatenate([s[0] for s in sel_parts], axis=0)
    cs_enc = jnp.concatenate([s[1] for s in sel_parts], axis=0)
    sel_tok = jnp.concatenate([s[2] for s in sel_parts], axis=0)

    ctx_enc = jnp.concatenate(
        [p[0] for p in ctx_parts], axis=0).reshape(N, TS, D)
    full_enc = jnp.concatenate([cs_enc, ctx_enc], axis=1)
    cs_mask = sel_tok.reshape(N, TK) != 0
    full_mask = jnp.concatenate([cs_mask, src_tokens != 0], axis=1)
    return full_enc, full_mask, ck_attn3.reshape(N, K)


# speedup vs baseline: 1.0447x; 1.0447x over previous
import functools
import math

import jax
import jax.numpy as jnp
import numpy as np
from jax import lax
from jax.experimental import pallas as pl
from jax.experimental.pallas import tpu as pltpu
from jax.experimental.pallas import tpu_sc as plsc

D = 256
L = 2
H = 4
DH = D // H
DFF = 1024
N = 8
TS = 512
K = 16
TK = 128
NEG = -1e9


def _sinusoid_np(T, d):
    pos = np.arange(T)[:, None].astype(np.float32)
    i = np.arange(d)[None, :].astype(np.float32)
    angle = pos / np.power(10000.0, (2.0 * np.floor(i / 2.0)) / d)
    pe = np.zeros((T, d), dtype=np.float32)
    pe[:, 0::2] = np.sin(angle[:, 0::2])
    pe[:, 1::2] = np.cos(angle[:, 1::2])
    return pe


_PE_CTX = _sinusoid_np(TS, D)
_PE_KN8 = np.tile(_sinusoid_np(TK, D), (8, 1))


def _sc_embed_gather(table, idx):
    info = plsc.get_sparse_core_info()
    nw = info.num_cores * info.num_subcores
    b = idx.shape[0]
    b_per_w = b // nw
    ch = max(c for c in range(1, min(128, b_per_w) + 1)
             if b_per_w % c == 0)
    n_ch = b_per_w // ch
    nc = info.num_cores
    NB = min(3, n_ch)
    idx3d = idx.reshape(nw, n_ch, ch)
    mesh = plsc.VectorSubcoreMesh(core_axis_name="c", subcore_axis_name="s")

    @functools.partial(
        pl.kernel,
        mesh=mesh,
        out_type=jax.ShapeDtypeStruct((b, D), jnp.float32),
        scratch_types=[
            pltpu.VMEM((n_ch, ch), jnp.int32),
            pltpu.VMEM((NB, ch, D), jnp.float32),
            pltpu.SemaphoreType.DMA,
            pltpu.SemaphoreType.DMA,
        ],
    )
    def gather_kernel(table_hbm, idx_hbm, out_hbm, idx_v, rows_v, gsem, wsem):
        wid = lax.axis_index("s") * nc + lax.axis_index("c")
        base = wid * b_per_w
        pltpu.sync_copy(idx_hbm.at[wid], idx_v)
        gathers = [None] * n_ch
        writes = [None] * n_ch
        for i in range(min(NB, n_ch)):
            gathers[i] = pltpu.async_copy(
                table_hbm.at[idx_v.at[i]], rows_v.at[i], gsem)
        for i in range(n_ch):
            gathers[i].wait()
            writes[i] = pltpu.async_copy(
                rows_v.at[i % NB], out_hbm.at[pl.ds(base + i * ch, ch)], wsem)
            if i + NB < n_ch:
                writes[i].wait()
                gathers[i + NB] = pltpu.async_copy(
                    table_hbm.at[idx_v.at[i + NB]], rows_v.at[i % NB], gsem)
        for i in range(max(0, n_ch - NB), n_ch):
            writes[i].wait()

    return gather_kernel(table, idx3d)


def _ln(x, g, b):
    mu = jnp.mean(x, axis=-1, keepdims=True)
    m2 = jnp.mean(x * x, axis=-1, keepdims=True)
    var = m2 - mu * mu
    return (x - mu) * lax.rsqrt(var + 1e-5) * g + b


def _encode_body(emb_ref, mrow_ref, mcol_ref, pe_ref,
                 wq_ref, wk_ref, wv_ref, wo_ref,
                 w1_ref, b1_ref, w2_ref, b2_ref,
                 g1_ref, bn1_ref, g2_ref, bn2_ref,
                 enc_ref, pool_ref, *, T, G):
    gt = G * T
    mrow = mrow_ref[0]
    mcol = mcol_ref[0]
    bias = (mrow - 1.0) * 1e9
    x = emb_ref[0] * np.float32(math.sqrt(D)) + pe_ref[...]
    inv_sqrt_dh = np.float32(1.0 / math.sqrt(DH))
    for l in range(L):
        q = jnp.dot(x, wq_ref[l])
        k = jnp.dot(x, wk_ref[l])
        v = jnp.dot(x, wv_ref[l])
        wo = wo_ref[l]
        o_rows = []
        for g in range(G):
            rs = slice(g * T, (g + 1) * T)
            bias_g = bias[g:g + 1, :]
            acc = None
            for h in range(H):
                cs = slice(h * DH, (h + 1) * DH)
                s = lax.dot_general(q[rs, cs], k[rs, cs],
                                    (((1,), (1,)), ((), ()))) * inv_sqrt_dh
                s = s + bias_g
                s = s - jnp.max(s, axis=-1, keepdims=True)
                p = jnp.exp(s)
                denom = jnp.sum(p, axis=-1, keepdims=True)
                oh = jnp.dot(p, v[rs, cs]) / denom
                part = jnp.dot(oh, wo[cs, :])
                acc = part if acc is None else acc + part
            o_rows.append(acc)
        o = jnp.concatenate(o_rows, axis=0) if G > 1 else o_rows[0]
        x = _ln(x + o, g1_ref[l], bn1_ref[l])
        hdn = jnp.maximum(jnp.dot(x, w1_ref[l]) + b1_ref[l], 0.0)
        x = _ln(x + jnp.dot(hdn, w2_ref[l]) + b2_ref[l], g2_ref[l], bn2_ref[l])
    xm = x * mcol
    enc_ref[0] = xm
    for g in range(G):
        seg = xm[g * T:(g + 1) * T, :]
        ssum = jnp.sum(seg, axis=0, keepdims=True)
        ln_g = jnp.maximum(jnp.sum(mrow[g]), 1.0)
        pool_ref[0, g:g + 1, :] = ssum * lax.rsqrt(ln_g * np.float32(D))


def _encode_pallas(emb3d, mrow, mcol, pe_big,
                   Wq, Wk, Wv, Wo, W1, b1, W2, b2, g1, bn1, g2, bn2,
                   *, T, G):
    nblk = emb3d.shape[0]
    gt = G * T
    full = lambda shape: pl.BlockSpec(shape, lambda i: tuple(0 for _ in shape))
    out = pl.pallas_call(
        functools.partial(_encode_body, T=T, G=G),
        grid=(nblk,),
        in_specs=[
            pl.BlockSpec((1, gt, D), lambda i: (i, 0, 0)),
            pl.BlockSpec((1, G, T), lambda i: (i, 0, 0)),
            pl.BlockSpec((1, gt, 1), lambda i: (i, 0, 0)),
            full((gt, D)),
            full((L, D, D)), full((L, D, D)), full((L, D, D)), full((L, D, D)),
            full((L, D, DFF)), full((L, DFF)),
            full((L, DFF, D)), full((L, D)),
            full((L, D)), full((L, D)), full((L, D)), full((L, D)),
        ],
        out_specs=[
            pl.BlockSpec((1, gt, D), lambda i: (i, 0, 0)),
            pl.BlockSpec((1, G, D), lambda i: (i, 0, 0)),
        ],
        out_shape=[
            jax.ShapeDtypeStruct((nblk, gt, D), jnp.float32),
            jax.ShapeDtypeStruct((nblk, G, D), jnp.float32),
        ],
    )(emb3d, mrow, mcol, pe_big, Wq, Wk, Wv, Wo, W1, b1, W2, b2,
      g1, bn1, g2, bn2)
    return out


def _select_body(ids_ref, flag_ref,
                 ku_ref, cu_ref, ckm_ref, ke_ref, kt_ref,
                 attn_ref, cse_ref, stok_ref):
    n = pl.program_id(0)
    ku = ku_ref[0]
    cu = cu_ref[0]
    raw = lax.dot_general(cu, ku, (((1,), (1,)), ((), ())))
    ckm = ckm_ref[0]
    attn_ref[0] = raw * ckm
    masked = jnp.where(ckm > 0.0, raw, NEG)
    mx = jnp.max(masked)
    iota = lax.broadcasted_iota(jnp.int32, (1, K), 1)
    amax = jnp.min(jnp.where(masked >= mx, iota, K))
    sel = jnp.where(flag_ref[0] != 0, ids_ref[n], amax)
    cse_ref[...] = ke_ref[0, pl.ds(sel, 1)]
    stok_ref[0] = kt_ref[0, pl.ds(sel, 1)]


def _select_pallas(cs_ids, use_flag, know_use, ctx_use, ckm, know_enc, know_tok):
    nb = know_use.shape[0]
    grid_spec = pltpu.PrefetchScalarGridSpec(
        num_scalar_prefetch=2,
        grid=(nb,),
        in_specs=[
            pl.BlockSpec((1, K, D), lambda n, i, f: (n, 0, 0)),
            pl.BlockSpec((1, 1, D), lambda n, i, f: (n, 0, 0)),
            pl.BlockSpec((1, 1, K), lambda n, i, f: (n, 0, 0)),
            pl.BlockSpec((1, K, TK, D), lambda n, i, f: (n, 0, 0, 0)),
            pl.BlockSpec((1, K, TK), lambda n, i, f: (n, 0, 0)),
        ],
        out_specs=[
            pl.BlockSpec((1, 1, K), lambda n, i, f: (n, 0, 0)),
            pl.BlockSpec((1, TK, D), lambda n, i, f: (n, 0, 0)),
            pl.BlockSpec((1, 1, TK), lambda n, i, f: (n, 0, 0)),
        ],
    )
    return pl.pallas_call(
        _select_body,
        grid_spec=grid_spec,
        out_shape=[
            jax.ShapeDtypeStruct((nb, 1, K), jnp.float32),
            jax.ShapeDtypeStruct((nb, TK, D), jnp.float32),
            jax.ShapeDtypeStruct((nb, 1, TK), jnp.int32),
        ],
    )(cs_ids, use_flag, know_use, ctx_use, ckm, know_enc, know_tok)


def kernel(src_tokens, know_tokens, ck_mask, cs_ids, use_cs_ids, embed,
           Wq, Wk, Wv, Wo, W1, b1, W2, b2, g1, bn1, g2, bn2):
    src_tokens = src_tokens.astype(jnp.int32)
    know_tokens = know_tokens.astype(jnp.int32)
    kn_splits = [(0, 3), (3, 6), (6, 7), (7, 8)]
    ctx_splits = [(0, 4), (4, 8)]
    ctx_embs = [
        _sc_embed_gather(embed, src_tokens[lo:hi].reshape(-1))
        .reshape(-1, TS, D) for lo, hi in ctx_splits]
    kn_embs = [
        _sc_embed_gather(embed, know_tokens[lo:hi].reshape(-1))
        .reshape(-1, 8 * TK, D) for lo, hi in kn_splits]

    ctx_maskf = (src_tokens != 0).astype(jnp.float32)
    kn_flat = know_tokens.reshape(N * K, TK)
    kn_maskf = (kn_flat != 0).astype(jnp.float32)

    weights = (Wq, Wk, Wv, Wo, W1, b1, W2, b2, g1, bn1, g2, bn2)
    ctx_mrow = ctx_maskf.reshape(N, 1, TS)
    ctx_mcol = ctx_maskf.reshape(N, TS, 1)
    ctx_parts = [
        _encode_pallas(ctx_embs[j], ctx_mrow[lo:hi],
                       ctx_mcol[lo:hi], _PE_CTX,
                       *weights, T=TS, G=1)
        for j, (lo, hi) in enumerate(ctx_splits)]
    kn_mrow = kn_maskf.reshape(16, 8, TK)
    kn_mcol = kn_maskf.reshape(16, 8 * TK, 1)
    kn_parts = [
        _encode_pallas(kn_embs[j], kn_mrow[2 * lo:2 * hi],
                       kn_mcol[2 * lo:2 * hi], _PE_KN8,
                       *weights, T=TK, G=8)
        for j, (lo, hi) in enumerate(kn_splits)]
    ctx_pool = jnp.concatenate([p[1] for p in ctx_parts], axis=0)

    ctx_use = ctx_pool.reshape(N, 1, D)
    ckm = ck_mask.astype(jnp.float32).reshape(N, 1, K)
    flag = jnp.asarray(use_cs_ids, jnp.int32).reshape(1)
    ids = cs_ids.astype(jnp.int32)
    know_tok4 = know_tokens.reshape(N, K, TK)

    sel_parts = [
        _select_pallas(
            ids[lo:hi], flag,
            kn_parts[j][1].reshape(-1, K, D),
            ctx_use[lo:hi],
            ckm[lo:hi],
            kn_parts[j][0].reshape(-1, K, TK, D),
            know_tok4[lo:hi])
        for j, (lo, hi) in enumerate(kn_splits)]
    ck_attn3 = jnp.concatenate([s[0] for s in sel_parts], axis=0)
    cs_enc = jnp.concatenate([s[1] for s in sel_parts], axis=0)
    sel_tok = jnp.concatenate([s[2] for s in sel_parts], axis=0)

    ctx_enc = jnp.concatenate(
        [p[0] for p in ctx_parts], axis=0).reshape(N, TS, D)
    full_enc = jnp.concatenate([cs_enc, ctx_enc], axis=1)
    cs_mask = sel_tok.reshape(N, TK) != 0
    full_mask = jnp.concatenate([cs_mask, src_tokens != 0], axis=1)
    return full_enc, full_mask, ck_attn3.reshape(N, K)


# speedup vs baseline: 1.2014x; 1.1500x over previous
import functools
import math

import jax
import jax.numpy as jnp
import numpy as np
from jax import lax
from jax.experimental import pallas as pl
from jax.experimental.pallas import tpu as pltpu
from jax.experimental.pallas import tpu_sc as plsc

D = 256
L = 2
H = 4
DH = D // H
DFF = 1024
N = 8
TS = 512
K = 16
TK = 128
NEG = -1e9


def _sinusoid_np(T, d):
    pos = np.arange(T)[:, None].astype(np.float32)
    i = np.arange(d)[None, :].astype(np.float32)
    angle = pos / np.power(10000.0, (2.0 * np.floor(i / 2.0)) / d)
    pe = np.zeros((T, d), dtype=np.float32)
    pe[:, 0::2] = np.sin(angle[:, 0::2])
    pe[:, 1::2] = np.cos(angle[:, 1::2])
    return pe


_PE_CTX = _sinusoid_np(TS, D)
_PE_KN8 = np.tile(_sinusoid_np(TK, D), (8, 1))


def _sc_embed_gather(table, idx):
    info = plsc.get_sparse_core_info()
    nw = info.num_cores * info.num_subcores
    b = idx.shape[0]
    b_per_w = b // nw
    ch = max(c for c in range(1, min(128, b_per_w) + 1)
             if b_per_w % c == 0)
    n_ch = b_per_w // ch
    nc = info.num_cores
    NB = min(3, n_ch)
    idx3d = idx.reshape(nw, n_ch, ch)
    mesh = plsc.VectorSubcoreMesh(core_axis_name="c", subcore_axis_name="s")

    @functools.partial(
        pl.kernel,
        mesh=mesh,
        out_type=jax.ShapeDtypeStruct((b, D), jnp.float32),
        scratch_types=[
            pltpu.VMEM((n_ch, ch), jnp.int32),
            pltpu.VMEM((NB, ch, D), jnp.float32),
            pltpu.SemaphoreType.DMA,
            pltpu.SemaphoreType.DMA,
        ],
    )
    def gather_kernel(table_hbm, idx_hbm, out_hbm, idx_v, rows_v, gsem, wsem):
        wid = lax.axis_index("s") * nc + lax.axis_index("c")
        base = wid * b_per_w
        pltpu.sync_copy(idx_hbm.at[wid], idx_v)
        gathers = [None] * n_ch
        writes = [None] * n_ch
        for i in range(min(NB, n_ch)):
            gathers[i] = pltpu.async_copy(
                table_hbm.at[idx_v.at[i]], rows_v.at[i], gsem)
        for i in range(n_ch):
            gathers[i].wait()
            writes[i] = pltpu.async_copy(
                rows_v.at[i % NB], out_hbm.at[pl.ds(base + i * ch, ch)], wsem)
            if i + NB < n_ch:
                writes[i].wait()
                gathers[i + NB] = pltpu.async_copy(
                    table_hbm.at[idx_v.at[i + NB]], rows_v.at[i % NB], gsem)
        for i in range(max(0, n_ch - NB), n_ch):
            writes[i].wait()

    return gather_kernel(table, idx3d)


def _ln(x, g, b):
    mu = jnp.mean(x, axis=-1, keepdims=True)
    m2 = jnp.mean(x * x, axis=-1, keepdims=True)
    var = m2 - mu * mu
    return (x - mu) * lax.rsqrt(var + 1e-5) * g + b


def _encode_body(emb_ref, mrow_ref, mcol_ref, pe_ref,
                 wq_ref, wk_ref, wv_ref, wo_ref,
                 w1_ref, b1_ref, w2_ref, b2_ref,
                 g1_ref, bn1_ref, g2_ref, bn2_ref,
                 enc_ref, pool_ref, *, T, G):
    gt = G * T
    mrow = mrow_ref[0]
    mcol = mcol_ref[0]
    bias = (mrow - 1.0) * 1e9
    x = emb_ref[0] * np.float32(math.sqrt(D)) + pe_ref[...]
    inv_sqrt_dh = np.float32(1.0 / math.sqrt(DH))
    for l in range(L):
        q = jnp.dot(x, wq_ref[l])
        k = jnp.dot(x, wk_ref[l])
        v = jnp.dot(x, wv_ref[l])
        wo = wo_ref[l]
        o_rows = []
        for g in range(G):
            rs = slice(g * T, (g + 1) * T)
            bias_g = bias[g:g + 1, :]
            acc = None
            for h in range(H):
                cs = slice(h * DH, (h + 1) * DH)
                s = lax.dot_general(q[rs, cs], k[rs, cs],
                                    (((1,), (1,)), ((), ()))) * inv_sqrt_dh
                s = s + bias_g
                p = jnp.exp(s)
                denom = jnp.sum(p, axis=-1, keepdims=True)
                oh = jnp.dot(p, v[rs, cs]) / denom
                part = jnp.dot(oh, wo[cs, :])
                acc = part if acc is None else acc + part
            o_rows.append(acc)
        o = jnp.concatenate(o_rows, axis=0) if G > 1 else o_rows[0]
        x = _ln(x + o, g1_ref[l], bn1_ref[l])
        hdn = jnp.maximum(jnp.dot(x, w1_ref[l]) + b1_ref[l], 0.0)
        x = _ln(x + jnp.dot(hdn, w2_ref[l]) + b2_ref[l], g2_ref[l], bn2_ref[l])
    xm = x * mcol
    enc_ref[0] = xm
    for g in range(G):
        seg = xm[g * T:(g + 1) * T, :]
        ssum = jnp.sum(seg, axis=0, keepdims=True)
        ln_g = jnp.maximum(jnp.sum(mrow[g]), 1.0)
        pool_ref[0, g:g + 1, :] = ssum * lax.rsqrt(ln_g * np.float32(D))


def _encode_pallas(emb3d, mrow, mcol, pe_big,
                   Wq, Wk, Wv, Wo, W1, b1, W2, b2, g1, bn1, g2, bn2,
                   *, T, G):
    nblk = emb3d.shape[0]
    gt = G * T
    full = lambda shape: pl.BlockSpec(shape, lambda i: tuple(0 for _ in shape))
    out = pl.pallas_call(
        functools.partial(_encode_body, T=T, G=G),
        grid=(nblk,),
        in_specs=[
            pl.BlockSpec((1, gt, D), lambda i: (i, 0, 0)),
            pl.BlockSpec((1, G, T), lambda i: (i, 0, 0)),
            pl.BlockSpec((1, gt, 1), lambda i: (i, 0, 0)),
            full((gt, D)),
            full((L, D, D)), full((L, D, D)), full((L, D, D)), full((L, D, D)),
            full((L, D, DFF)), full((L, DFF)),
            full((L, DFF, D)), full((L, D)),
            full((L, D)), full((L, D)), full((L, D)), full((L, D)),
        ],
        out_specs=[
            pl.BlockSpec((1, gt, D), lambda i: (i, 0, 0)),
            pl.BlockSpec((1, G, D), lambda i: (i, 0, 0)),
        ],
        out_shape=[
            jax.ShapeDtypeStruct((nblk, gt, D), jnp.float32),
            jax.ShapeDtypeStruct((nblk, G, D), jnp.float32),
        ],
    )(emb3d, mrow, mcol, pe_big, Wq, Wk, Wv, Wo, W1, b1, W2, b2,
      g1, bn1, g2, bn2)
    return out


def _select_body(ids_ref, flag_ref,
                 ku_ref, cu_ref, ckm_ref, ke_ref, kt_ref,
                 attn_ref, cse_ref, stok_ref):
    n = pl.program_id(0)
    ku = ku_ref[0]
    cu = cu_ref[0]
    raw = lax.dot_general(cu, ku, (((1,), (1,)), ((), ())))
    ckm = ckm_ref[0]
    attn_ref[0] = raw * ckm
    masked = jnp.where(ckm > 0.0, raw, NEG)
    mx = jnp.max(masked)
    iota = lax.broadcasted_iota(jnp.int32, (1, K), 1)
    amax = jnp.min(jnp.where(masked >= mx, iota, K))
    sel = jnp.where(flag_ref[0] != 0, ids_ref[n], amax)
    cse_ref[...] = ke_ref[0, pl.ds(sel, 1)]
    stok_ref[0] = kt_ref[0, pl.ds(sel, 1)]


def _select_pallas(cs_ids, use_flag, know_use, ctx_use, ckm, know_enc, know_tok):
    nb = know_use.shape[0]
    grid_spec = pltpu.PrefetchScalarGridSpec(
        num_scalar_prefetch=2,
        grid=(nb,),
        in_specs=[
            pl.BlockSpec((1, K, D), lambda n, i, f: (n, 0, 0)),
            pl.BlockSpec((1, 1, D), lambda n, i, f: (n, 0, 0)),
            pl.BlockSpec((1, 1, K), lambda n, i, f: (n, 0, 0)),
            pl.BlockSpec((1, K, TK, D), lambda n, i, f: (n, 0, 0, 0)),
            pl.BlockSpec((1, K, TK), lambda n, i, f: (n, 0, 0)),
        ],
        out_specs=[
            pl.BlockSpec((1, 1, K), lambda n, i, f: (n, 0, 0)),
            pl.BlockSpec((1, TK, D), lambda n, i, f: (n, 0, 0)),
            pl.BlockSpec((1, 1, TK), lambda n, i, f: (n, 0, 0)),
        ],
    )
    return pl.pallas_call(
        _select_body,
        grid_spec=grid_spec,
        out_shape=[
            jax.ShapeDtypeStruct((nb, 1, K), jnp.float32),
            jax.ShapeDtypeStruct((nb, TK, D), jnp.float32),
            jax.ShapeDtypeStruct((nb, 1, TK), jnp.int32),
        ],
    )(cs_ids, use_flag, know_use, ctx_use, ckm, know_enc, know_tok)


def kernel(src_tokens, know_tokens, ck_mask, cs_ids, use_cs_ids, embed,
           Wq, Wk, Wv, Wo, W1, b1, W2, b2, g1, bn1, g2, bn2):
    src_tokens = src_tokens.astype(jnp.int32)
    know_tokens = know_tokens.astype(jnp.int32)
    kn_splits = [(0, 3), (3, 6), (6, 7), (7, 8)]
    ctx_splits = [(0, 4), (4, 8)]
    ctx_embs = [
        _sc_embed_gather(embed, src_tokens[lo:hi].reshape(-1))
        .reshape(-1, TS, D) for lo, hi in ctx_splits]
    kn_embs = [
        _sc_embed_gather(embed, know_tokens[lo:hi].reshape(-1))
        .reshape(-1, 8 * TK, D) for lo, hi in kn_splits]

    ctx_maskf = (src_tokens != 0).astype(jnp.float32)
    kn_flat = know_tokens.reshape(N * K, TK)
    kn_maskf = (kn_flat != 0).astype(jnp.float32)

    weights = (Wq, Wk, Wv, Wo, W1, b1, W2, b2, g1, bn1, g2, bn2)
    ctx_mrow = ctx_maskf.reshape(N, 1, TS)
    ctx_mcol = ctx_maskf.reshape(N, TS, 1)
    ctx_parts = [
        _encode_pallas(ctx_embs[j], ctx_mrow[lo:hi],
                       ctx_mcol[lo:hi], _PE_CTX,
                       *weights, T=TS, G=1)
        for j, (lo, hi) in enumerate(ctx_splits)]
    kn_mrow = kn_maskf.reshape(16, 8, TK)
    kn_mcol = kn_maskf.reshape(16, 8 * TK, 1)
    kn_parts = [
        _encode_pallas(kn_embs[j], kn_mrow[2 * lo:2 * hi],
                       kn_mcol[2 * lo:2 * hi], _PE_KN8,
                       *weights, T=TK, G=8)
        for j, (lo, hi) in enumerate(kn_splits)]
    ctx_pool = jnp.concatenate([p[1] for p in ctx_parts], axis=0)

    ctx_use = ctx_pool.reshape(N, 1, D)
    ckm = ck_mask.astype(jnp.float32).reshape(N, 1, K)
    flag = jnp.asarray(use_cs_ids, jnp.int32).reshape(1)
    ids = cs_ids.astype(jnp.int32)
    know_tok4 = know_tokens.reshape(N, K, TK)

    sel_parts = [
        _select_pallas(
            ids[lo:hi], flag,
            kn_parts[j][1].reshape(-1, K, D),
            ctx_use[lo:hi],
            ckm[lo:hi],
            kn_parts[j][0].reshape(-1, K, TK, D),
            know_tok4[lo:hi])
        for j, (lo, hi) in enumerate(kn_splits)]
    ck_attn3 = jnp.concatenate([s[0] for s in sel_parts], axis=0)
    cs_enc = jnp.concatenate([s[1] for s in sel_parts], axis=0)
    sel_tok = jnp.concatenate([s[2] for s in sel_parts], axis=0)

    ctx_enc = jnp.concatenate(
        [p[0] for p in ctx_parts], axis=0).reshape(N, TS, D)
    full_enc = jnp.concatenate([cs_enc, ctx_enc], axis=1)
    cs_mask = sel_tok.reshape(N, TK) != 0
    full_mask = jnp.concatenate([cs_mask, src_tokens != 0], axis=1)
    return full_enc, full_mask, ck_attn3.reshape(N, K)
